# baseline (speedup 1.0000x reference)
import functools

import jax
import jax.numpy as jnp
from jax import lax
from jax.experimental import pallas as pl
from jax.experimental.pallas import tpu as pltpu
from jax.experimental.pallas import tpu_sc as plsc

_NC = 2
_NS = 16
_L = 16
_NW = _NC * _NS
_NCHUNKS = 8


@functools.cache
def _sc_pair_lookup_call(n: int, conv_words: int):
    n_per_w = n // _NW
    assert n_per_w % (_NCHUNKS * _L) == 0, n
    c = n_per_w // _NCHUNKS
    c2 = 2 * c
    mesh = plsc.VectorSubcoreMesh(core_axis_name="c", subcore_axis_name="s")

    @functools.partial(
        pl.kernel,
        out_type=jax.ShapeDtypeStruct((n,), jnp.int32),
        mesh=mesh,
        scratch_types=[
            pltpu.VMEM((conv_words,), jnp.int32),
            pltpu.VMEM((c2,), jnp.int32),
            pltpu.VMEM((c2,), jnp.int32),
            pltpu.VMEM((c,), jnp.int32),
            pltpu.VMEM((c,), jnp.int32),
            pltpu.SemaphoreType.DMA,
            pltpu.SemaphoreType.DMA,
            pltpu.SemaphoreType.DMA,
            pltpu.SemaphoreType.DMA,
        ],
        compiler_params=pltpu.CompilerParams(needs_layout_passes=False),
    )
    def body(sp_hbm, conv_hbm, out_hbm, conv_v, in0, in1, out0, out1,
             si0, si1, so0, so1):
        wid = lax.axis_index("s") * jnp.int32(_NC) + lax.axis_index("c")
        base2 = wid * jnp.int32(2 * n_per_w)
        base = wid * jnp.int32(n_per_w)
        ins, outs = (in0, in1), (out0, out1)
        isems, osems = (si0, si1), (so0, so1)

        def in_copy(k):
            return pltpu.make_async_copy(
                sp_hbm.at[pl.ds(base2 + k * c2, c2)], ins[k % 2], isems[k % 2])

        def out_copy(k):
            return pltpu.make_async_copy(
                outs[k % 2], out_hbm.at[pl.ds(base + k * c, c)], osems[k % 2])

        in_copy(0).start()
        in_copy(1).start()
        pltpu.sync_copy(conv_hbm, conv_v)
        iota2 = lax.iota(jnp.int32, _L) * jnp.int32(2)

        for k in range(_NCHUNKS):
            in_copy(k).wait()
            if k >= 2:
                out_copy(k - 2).wait()
            ib, ob = ins[k % 2], outs[k % 2]

            @plsc.parallel_loop(jnp.int32(0), jnp.int32(c),
                                step=jnp.int32(_L), unroll=8)
            def _(off):
                lo = plsc.load_gather(ib, [iota2 + jnp.int32(2) * off])
                ob[pl.ds(off, _L)] = plsc.load_gather(conv_v, [lo])

            out_copy(k).start()
            if k + 2 < _NCHUNKS:
                in_copy(k + 2).start()

        out_copy(_NCHUNKS - 2).wait()
        out_copy(_NCHUNKS - 1).wait()

    return body


def kernel(species, coordinates, conv_tensor):
    shape = species.shape
    n = species.size
    conv16 = (
        jnp.zeros((_L,), jnp.int32)
        .at[: conv_tensor.shape[0]]
        .set(conv_tensor.astype(jnp.int32))
    )
    if species.dtype.itemsize == 8:
        sp = lax.bitcast_convert_type(species, jnp.int32).reshape(2 * n)
        out32 = _sc_pair_lookup_call(n, _L)(sp, conv16)
    else:
        out32 = _sc_pair_lookup_call(n, _L)(
            jnp.stack([species.astype(jnp.int32)] * 2, -1).reshape(2 * n),
            conv16)
    return out32.reshape(shape).astype(conv_tensor.dtype), coordinates


# speedup vs baseline: 8.3158x; 8.3158x over previous
import functools

import jax
import jax.numpy as jnp
from jax import lax
from jax.experimental import pallas as pl
from jax.experimental.pallas import tpu as pltpu
from jax.experimental.pallas import tpu_sc as plsc

_NC = 2
_NS = 16
_L = 16
_NW = _NC * _NS
_NCHUNKS = 8


@functools.cache
def _sc_lookup_call(n: int, conv_words: int):
    n_per_w = n // _NW
    assert n_per_w % (_NCHUNKS * _L) == 0, n
    c = n_per_w // _NCHUNKS
    mesh = plsc.VectorSubcoreMesh(core_axis_name="c", subcore_axis_name="s")

    @functools.partial(
        pl.kernel,
        out_type=jax.ShapeDtypeStruct((n,), jnp.int32),
        mesh=mesh,
        scratch_types=[
            pltpu.VMEM((conv_words,), jnp.int32),
            pltpu.VMEM((c,), jnp.int32),
            pltpu.VMEM((c,), jnp.int32),
            pltpu.VMEM((c,), jnp.int32),
            pltpu.VMEM((c,), jnp.int32),
            pltpu.SemaphoreType.DMA,
            pltpu.SemaphoreType.DMA,
            pltpu.SemaphoreType.DMA,
            pltpu.SemaphoreType.DMA,
        ],
        compiler_params=pltpu.CompilerParams(needs_layout_passes=False),
    )
    def body(sp_hbm, conv_hbm, out_hbm, conv_v, in0, in1, out0, out1,
             si0, si1, so0, so1):
        wid = lax.axis_index("s") * jnp.int32(_NC) + lax.axis_index("c")
        base = wid * jnp.int32(n_per_w)
        ins, outs = (in0, in1), (out0, out1)
        isems, osems = (si0, si1), (so0, so1)

        def in_copy(k):
            return pltpu.make_async_copy(
                sp_hbm.at[pl.ds(base + k * c, c)], ins[k % 2], isems[k % 2])

        def out_copy(k):
            return pltpu.make_async_copy(
                outs[k % 2], out_hbm.at[pl.ds(base + k * c, c)], osems[k % 2])

        in_copy(0).start()
        in_copy(1).start()
        pltpu.sync_copy(conv_hbm, conv_v)

        for k in range(_NCHUNKS):
            in_copy(k).wait()
            if k >= 2:
                out_copy(k - 2).wait()
            ib, ob = ins[k % 2], outs[k % 2]

            @plsc.parallel_loop(jnp.int32(0), jnp.int32(c),
                                step=jnp.int32(_L), unroll=8)
            def _(off):
                idx = ib[pl.ds(off, _L)]
                ob[pl.ds(off, _L)] = plsc.load_gather(conv_v, [idx])

            out_copy(k).start()
            if k + 2 < _NCHUNKS:
                in_copy(k + 2).start()

        out_copy(_NCHUNKS - 2).wait()
        out_copy(_NCHUNKS - 1).wait()

    return body


def kernel(species, coordinates, conv_tensor):
    shape = species.shape
    n = species.size
    conv16 = (
        jnp.zeros((_L,), jnp.int32)
        .at[: conv_tensor.shape[0]]
        .set(conv_tensor.astype(jnp.int32))
    )
    sp32 = species.reshape(n).astype(jnp.int32)
    out32 = _sc_lookup_call(n, _L)(sp32, conv16)
    return out32.reshape(shape).astype(conv_tensor.dtype), coordinates
